# baseline (speedup 1.0000x reference)
import functools

import numpy as np
import jax
import jax.numpy as jnp
from jax import lax
from jax.experimental import pallas as pl
from jax.experimental.pallas import tpu as pltpu
from jax.experimental.pallas import tpu_sc as plsc

NUM_EMB = 1000000
EMB_DIM = 64
NUM_LEAVES = 16384
NUM_PAIRS = NUM_LEAVES // 2
NUM_CAT = 5

_NC = 2
_NS = 16
_NW = _NC * _NS
_B_PER_W = NUM_LEAVES // _NW

_ang = 2.0 * np.pi * np.outer(np.arange(EMB_DIM), np.arange(EMB_DIM)) / EMB_DIM
_C_MAT = np.cos(_ang).astype(np.float32)
_S_MAT = np.sin(_ang).astype(np.float32)


@functools.cache
def _make_sc_gather():
    @functools.partial(
        pl.kernel,
        out_type=jax.ShapeDtypeStruct((NUM_LEAVES, EMB_DIM), jnp.float32),
        mesh=plsc.VectorSubcoreMesh(
            core_axis_name="c", subcore_axis_name="s",
            num_cores=_NC, num_subcores=_NS),
        scratch_types=[
            pltpu.VMEM((_B_PER_W,), jnp.int32),
            pltpu.VMEM((_B_PER_W, EMB_DIM), jnp.float32),
            pltpu.SemaphoreType.DMA,
        ],
        compiler_params=pltpu.CompilerParams(use_tc_tiling_on_sc=True,
                                             needs_layout_passes=False),
    )
    def _sc_gather(idx_hbm, table_hbm, out_hbm, idx_v, rows_v, sem):
        wid = lax.axis_index("s") * _NC + lax.axis_index("c")
        base = wid * _B_PER_W
        pltpu.sync_copy(idx_hbm.at[pl.ds(base, _B_PER_W)], idx_v)
        lanes = lax.iota(jnp.int32, 16)

        def fire(g, carry):
            v = idx_v[pl.ds(g * 16, 16)]
            for l in range(16):
                i = jnp.max(jnp.where(lanes == l, v, 0), axis=0)
                pltpu.async_copy(
                    table_hbm.at[pl.ds(i, 1)],
                    rows_v.at[pl.ds(g * 16 + l, 1)],
                    sem)
            return carry

        lax.fori_loop(0, _B_PER_W // 16, fire, 0)
        pltpu.make_async_copy(
            out_hbm.at[pl.ds(base, _B_PER_W)], rows_v, sem).wait()
        pltpu.sync_copy(rows_v, out_hbm.at[pl.ds(base, _B_PER_W)])

    return _sc_gather


_BM = 2048


def _tc_body(v_ref, c_ref, s_ref, w_ref, b_ref, leaf_ref, parent_ref):
    v = v_ref[...]
    nrm = jnp.sqrt(jnp.sum(v * v, axis=1, keepdims=True)) + 1e-12
    v = v / nrm
    q = pltpu.roll(v, _BM - 1, 0)
    cm = c_ref[...]
    sm = s_ref[...]

    def dot(a, b):
        return lax.dot_general(a, b, (((1,), (0,)), ((), ())),
                               preferred_element_type=jnp.float32)

    pc = dot(v, cm)
    ps = dot(v, sm)
    qc = dot(q, cm)
    qs = dot(q, sm)
    zr = pc * qc + ps * qs
    zi = ps * qc - pc * qs
    parent = (dot(zr, cm) - dot(zi, sm)) * (1.0 / EMB_DIM)

    w = w_ref[...]
    bb = b_ref[...]

    def clf(x):
        logits = lax.dot_general(x, w, (((1,), (1,)), ((), ())),
                                 preferred_element_type=jnp.float32) + bb
        m = jnp.max(logits, axis=1, keepdims=True)
        e = jnp.exp(logits - m)
        return e / jnp.sum(e, axis=1, keepdims=True)

    leaf_ref[...] = clf(v)

    rows = lax.broadcasted_iota(jnp.int32, (_BM // 2, _BM), 0)
    cols = lax.broadcasted_iota(jnp.int32, (_BM // 2, _BM), 1)
    sel = (cols == 2 * rows).astype(jnp.float32)
    parent_sel = lax.dot_general(
        sel, parent, (((1,), (0,)), ((), ())),
        preferred_element_type=jnp.float32)
    parent_ref[...] = clf(parent_sel)


def _tc_dense(gathered, w, b2d):
    grid = NUM_LEAVES // _BM
    v_spec = pl.BlockSpec((_BM, EMB_DIM), lambda i: (i, 0))
    mat_spec = pl.BlockSpec((EMB_DIM, EMB_DIM), lambda i: (0, 0))
    w_spec = pl.BlockSpec((NUM_CAT, EMB_DIM), lambda i: (0, 0))
    b_spec = pl.BlockSpec((1, NUM_CAT), lambda i: (0, 0))
    leaf_spec = pl.BlockSpec((_BM, NUM_CAT), lambda i: (i, 0))
    parent_spec = pl.BlockSpec((_BM // 2, NUM_CAT), lambda i: (i, 0))
    return pl.pallas_call(
        _tc_body,
        grid=(grid,),
        in_specs=[v_spec, mat_spec, mat_spec, w_spec, b_spec],
        out_specs=[leaf_spec, parent_spec],
        out_shape=[
            jax.ShapeDtypeStruct((NUM_LEAVES, NUM_CAT), jnp.float32),
            jax.ShapeDtypeStruct((NUM_PAIRS, NUM_CAT), jnp.float32),
        ],
    )(gathered, _C_MAT, _S_MAT, w, b2d)


def kernel(indices, emb_table, W, b):
    gathered = _make_sc_gather()(indices, emb_table)
    leaf_p, parent_p = _tc_dense(gathered, W, b.reshape(1, NUM_CAT))
    return jnp.concatenate([leaf_p, parent_p], axis=0)


# speedup vs baseline: 1.0096x; 1.0096x over previous
import functools

import numpy as np
import jax
import jax.numpy as jnp
from jax import lax
from jax.experimental import pallas as pl
from jax.experimental.pallas import tpu as pltpu
from jax.experimental.pallas import tpu_sc as plsc

NUM_EMB = 1000000
EMB_DIM = 64
NUM_LEAVES = 16384
NUM_PAIRS = NUM_LEAVES // 2
NUM_CAT = 5

_NC = 2
_NS = 16
_NW = _NC * _NS
_B_PER_W = NUM_LEAVES // _NW

_ang = 2.0 * np.pi * np.outer(np.arange(EMB_DIM), np.arange(EMB_DIM)) / EMB_DIM
_C_MAT = np.cos(_ang).astype(np.float32)
_S_MAT = np.sin(_ang).astype(np.float32)


@functools.cache
def _make_sc_gather():
    @functools.partial(
        pl.kernel,
        out_type=jax.ShapeDtypeStruct((NUM_LEAVES, EMB_DIM), jnp.float32),
        mesh=plsc.VectorSubcoreMesh(
            core_axis_name="c", subcore_axis_name="s",
            num_cores=_NC, num_subcores=_NS),
        scratch_types=[
            pltpu.VMEM((_B_PER_W,), jnp.int32),
            pltpu.VMEM((_B_PER_W, EMB_DIM), jnp.float32),
            pltpu.SemaphoreType.DMA,
        ],
        compiler_params=pltpu.CompilerParams(use_tc_tiling_on_sc=True,
                                             needs_layout_passes=False),
    )
    def _sc_gather(idx_hbm, table_hbm, out_hbm, idx_v, rows_v, sem):
        wid = lax.axis_index("s") * _NC + lax.axis_index("c")
        base = wid * _B_PER_W
        pltpu.sync_copy(idx_hbm.at[pl.ds(base, _B_PER_W)], idx_v)
        lanes = lax.iota(jnp.int32, 16)

        def fire(g, carry):
            v = idx_v[pl.ds(g * 16, 16)]
            for l in range(16):
                i = jnp.max(jnp.where(lanes == l, v, 0), axis=0)
                pltpu.async_copy(
                    table_hbm.at[pl.ds(i, 1)],
                    rows_v.at[pl.ds(g * 16 + l, 1)],
                    sem)
            return carry

        lax.fori_loop(0, _B_PER_W // 16, fire, 0)
        pltpu.make_async_copy(
            out_hbm.at[pl.ds(base, _B_PER_W)], rows_v, sem).wait()
        pltpu.sync_copy(rows_v, out_hbm.at[pl.ds(base, _B_PER_W)])

    return _sc_gather


_BM = 1024


def _tc_body(v_ref, c_ref, s_ref, w_ref, b_ref, leaf_ref, parent_ref):
    v = v_ref[...]
    nrm = jnp.sqrt(jnp.sum(v * v, axis=1, keepdims=True)) + 1e-12
    v = v / nrm
    q = pltpu.roll(v, _BM - 1, 0)
    cm = c_ref[...]
    sm = s_ref[...]

    def dot(a, b):
        return lax.dot_general(a, b, (((1,), (0,)), ((), ())),
                               preferred_element_type=jnp.float32)

    pc = dot(v, cm)
    ps = dot(v, sm)
    qc = dot(q, cm)
    qs = dot(q, sm)
    zr = pc * qc + ps * qs
    zi = ps * qc - pc * qs
    parent = (dot(zr, cm) - dot(zi, sm)) * (1.0 / EMB_DIM)

    w = w_ref[...]
    bb = b_ref[...]

    def clf(x):
        logits = lax.dot_general(x, w, (((1,), (1,)), ((), ())),
                                 preferred_element_type=jnp.float32) + bb
        m = jnp.max(logits, axis=1, keepdims=True)
        e = jnp.exp(logits - m)
        return e / jnp.sum(e, axis=1, keepdims=True)

    leaf_ref[...] = clf(v)

    rows = lax.broadcasted_iota(jnp.int32, (_BM // 2, _BM), 0)
    cols = lax.broadcasted_iota(jnp.int32, (_BM // 2, _BM), 1)
    sel = (cols == 2 * rows).astype(jnp.float32)
    parent_sel = lax.dot_general(
        sel, parent, (((1,), (0,)), ((), ())),
        preferred_element_type=jnp.float32)
    parent_ref[...] = clf(parent_sel)


def _tc_dense(gathered, w, b2d):
    grid = NUM_LEAVES // _BM
    v_spec = pl.BlockSpec((_BM, EMB_DIM), lambda i: (i, 0))
    mat_spec = pl.BlockSpec((EMB_DIM, EMB_DIM), lambda i: (0, 0))
    w_spec = pl.BlockSpec((NUM_CAT, EMB_DIM), lambda i: (0, 0))
    b_spec = pl.BlockSpec((1, NUM_CAT), lambda i: (0, 0))
    leaf_spec = pl.BlockSpec((_BM, NUM_CAT), lambda i: (i, 0))
    parent_spec = pl.BlockSpec((_BM // 2, NUM_CAT), lambda i: (i, 0))
    return pl.pallas_call(
        _tc_body,
        grid=(grid,),
        in_specs=[v_spec, mat_spec, mat_spec, w_spec, b_spec],
        out_specs=[leaf_spec, parent_spec],
        out_shape=[
            jax.ShapeDtypeStruct((NUM_LEAVES, NUM_CAT), jnp.float32),
            jax.ShapeDtypeStruct((NUM_PAIRS, NUM_CAT), jnp.float32),
        ],
    )(gathered, _C_MAT, _S_MAT, w, b2d)


def kernel(indices, emb_table, W, b):
    gathered = _make_sc_gather()(indices, emb_table)
    leaf_p, parent_p = _tc_dense(gathered, W, b.reshape(1, NUM_CAT))
    return jnp.concatenate([leaf_p, parent_p], axis=0)
